# baseline (speedup 1.0000x reference)
import jax
import jax.numpy as jnp
from jax import lax
from jax.experimental import pallas as pl
from jax.experimental.pallas import tpu as pltpu
from jax.experimental.pallas import tpu_sc as plsc

N = 10000
E = 320000
D = 128
H = 64
C = 10
G = 64

NC = 2
NS = 16
NW = NC * NS
EPW = E // NW
K = 128
EPAD = 323584
NCHUNK = EPAD // (NW * K)
NPAD = 10240
RPS = NPAD // NS

BN = 2000


NBUF = 8
GDEPTH = 4
ZROWS = 40


def _seg_sum_body(y_hbm, src_hbm, dst_hbm, out_hbm,
                  src_big, dst_big, rows, buf_v, acc_sh, gs, ss):
    c = lax.axis_index("c")
    s = lax.axis_index("s")
    wid = s * NC + c
    rbase = s * RPS

    zv = jnp.zeros((16,), jnp.float32)

    def zbody(r, carry):
        for u in range(H // 16):
            buf_v[r, pl.ds(16 * u, 16)] = zv
        return carry

    lax.fori_loop(0, ZROWS, zbody, 0)
    for zz in range(RPS // ZROWS):
        pltpu.sync_copy(buf_v, acc_sh.at[pl.ds(rbase + zz * ZROWS, ZROWS)])

    pltpu.sync_copy(src_hbm.at[pl.ds(wid * NCHUNK, NCHUNK)], src_big)
    pltpu.sync_copy(dst_hbm.at[pl.ds(wid * NCHUNK, NCHUNK)], dst_big)
    plsc.subcore_barrier()

    def gather(ck, b):
        pltpu.async_copy(y_hbm.at[src_big.at[ck]], rows[b], gs[b])

    def wait_gather(b):
        pltpu.make_async_copy(y_hbm.at[src_big.at[0]], rows[b], gs[b]).wait()

    def scatter(ck, b):
        pltpu.async_copy(rows[b], acc_sh.at[dst_big.at[ck]], ss[b], add=True)

    def wait_scatter(b):
        pltpu.make_async_copy(rows[b], acc_sh.at[dst_big.at[0]], ss[b]).wait()

    def full_step(cc, b):
        wait_gather(b)
        scatter(cc, b)
        bp = (b + GDEPTH) % NBUF
        wait_scatter(bp)
        gather(cc + GDEPTH, bp)

    def step(cc, b):
        wait_gather(b)
        scatter(cc, b)
        if cc + GDEPTH < NCHUNK:
            bp = (b + GDEPTH) % NBUF
            if cc + GDEPTH >= NBUF:
                wait_scatter(bp)
            gather(cc + GDEPTH, bp)

    for b in range(GDEPTH):
        gather(b, b)
    for cc in range(NBUF):
        step(cc, cc)

    n_loop = (NCHUNK - GDEPTH - NBUF) // NBUF

    def body(j, carry):
        c0 = j * NBUF
        for u in range(NBUF):
            full_step(c0 + u, u)
        return carry

    lax.fori_loop(1, 1 + n_loop, body, 0)

    for cc in range(NBUF + n_loop * NBUF, NCHUNK):
        step(cc, cc % NBUF)
    for b in range(NBUF):
        wait_scatter(b)
    plsc.subcore_barrier()

    ov = out_hbm.at[:, pl.ds(H * c, H)]
    for zz in range(RPS // ZROWS):
        pltpu.sync_copy(acc_sh.at[pl.ds(rbase + zz * ZROWS, ZROWS)], buf_v)
        pltpu.sync_copy(buf_v, ov.at[pl.ds(rbase + zz * ZROWS, ZROWS)])


def _seg_sum(y, src2, dst2):
    f = pl.kernel(
        _seg_sum_body,
        out_type=jax.ShapeDtypeStruct((NPAD, 2 * H), jnp.float32),
        mesh=plsc.VectorSubcoreMesh(core_axis_name="c", subcore_axis_name="s",
                                    num_cores=NC, num_subcores=NS),
        scratch_types=[
            pltpu.VMEM((NCHUNK, K), jnp.int32),
            pltpu.VMEM((NCHUNK, K), jnp.int32),
            [pltpu.VMEM((K, H), jnp.float32) for _ in range(NBUF)],
            pltpu.VMEM((ZROWS, H), jnp.float32),
            pltpu.VMEM_SHARED((NPAD, H), jnp.float32),
            [pltpu.SemaphoreType.DMA for _ in range(NBUF)],
            [pltpu.SemaphoreType.DMA for _ in range(NBUF)],
        ],
        compiler_params=pltpu.CompilerParams(use_tc_tiling_on_sc=False),
    )
    return f(y, src2, dst2)


def _dot_t(a, w):
    return lax.dot_general(a, w, (((1,), (1,)), ((), ())),
                           preferred_element_type=jnp.float32)


def _pre_body(x_ref, wr_ref, wo_ref, b_ref, yr_ref):
    x = x_ref[...]
    y = _dot_t(x, wr_ref[...])
    r = _dot_t(x, wo_ref[...]) + b_ref[...]
    yr_ref[...] = jnp.concatenate([y, r], axis=1)


_pre = pl.pallas_call(
    _pre_body,
    grid=(N // BN,),
    in_specs=[
        pl.BlockSpec((BN, D), lambda i: (i, 0)),
        pl.BlockSpec((H, D), lambda i: (0, 0)),
        pl.BlockSpec((H, D), lambda i: (0, 0)),
        pl.BlockSpec((1, H), lambda i: (0, 0)),
    ],
    out_specs=pl.BlockSpec((BN, 2 * H), lambda i: (i, 0)),
    out_shape=jax.ShapeDtypeStruct((N, 2 * H), jnp.float32),
)


def _mid_body(p_ref, yr_ref, wr_ref, wo_ref, b_ref, yrn_ref):
    p = p_ref[...]
    h = jnp.maximum(p[:, :H] + p[:, H:] + yr_ref[...][:, H:], 0.0)
    y = _dot_t(h, wr_ref[...])
    rn = _dot_t(h, wo_ref[...]) + b_ref[...]
    yrn_ref[...] = jnp.concatenate([y, rn], axis=1)


_mid = pl.pallas_call(
    _mid_body,
    grid=(N // BN,),
    in_specs=[
        pl.BlockSpec((BN, 2 * H), lambda i: (i, 0)),
        pl.BlockSpec((BN, 2 * H), lambda i: (i, 0)),
        pl.BlockSpec((H, H), lambda i: (0, 0)),
        pl.BlockSpec((H, H), lambda i: (0, 0)),
        pl.BlockSpec((1, H), lambda i: (0, 0)),
    ],
    out_specs=pl.BlockSpec((BN, 2 * H), lambda i: (i, 0)),
    out_shape=jax.ShapeDtypeStruct((N, 2 * H), jnp.float32),
)


def _fin_body(p_ref, yr_ref, batch_ref, wl_ref, bl_ref, out_ref,
              acc_ref, cnt_ref):
    i = pl.program_id(0)

    @pl.when(i == 0)
    def _():
        acc_ref[...] = jnp.zeros_like(acc_ref)
        cnt_ref[...] = jnp.zeros_like(cnt_ref)

    p = p_ref[...]
    h = p[:, :H] + p[:, H:] + yr_ref[...][:, H:]
    b = batch_ref[...][0]
    gi = lax.broadcasted_iota(jnp.int32, (G, BN), 0)
    onehot = (gi == b).astype(jnp.float32)
    acc_ref[...] += lax.dot_general(
        onehot, h, (((1,), (0,)), ((), ())),
        preferred_element_type=jnp.float32)
    cnt_ref[...] += jnp.sum(onehot, axis=1, keepdims=True)

    @pl.when(i == pl.num_programs(0) - 1)
    def _():
        pool = acc_ref[...] / jnp.maximum(cnt_ref[...], 1.0)
        out_ref[...] = _dot_t(pool, wl_ref[...]) + bl_ref[...]


_fin = pl.pallas_call(
    _fin_body,
    grid=(N // BN,),
    in_specs=[
        pl.BlockSpec((BN, 2 * H), lambda i: (i, 0)),
        pl.BlockSpec((BN, 2 * H), lambda i: (i, 0)),
        pl.BlockSpec((1, 1, BN), lambda i: (i, 0, 0)),
        pl.BlockSpec((C, H), lambda i: (0, 0)),
        pl.BlockSpec((1, C), lambda i: (0, 0)),
    ],
    out_specs=pl.BlockSpec((G, C), lambda i: (0, 0)),
    out_shape=jax.ShapeDtypeStruct((G, C), jnp.float32),
    scratch_shapes=[pltpu.VMEM((G, H), jnp.float32),
                    pltpu.VMEM((G, 1), jnp.float32)],
)


def kernel(x, edge_index, batch, W1_rel, b1, W1_root, W2_rel, b2, W2_root,
           W3_rel, b3, W3_root, W_lin, b_lin):
    pad = EPAD - E
    src = jnp.concatenate(
        [edge_index[0] * 2, jnp.zeros((pad,), jnp.int32)]).reshape(
            EPAD // K, K)
    pad_dst = N + (jnp.arange(pad, dtype=jnp.int32) % (NPAD - N))
    dst = jnp.concatenate([edge_index[1], pad_dst]).reshape(EPAD // K, K)
    b1r = b1.reshape(1, H)
    b2r = b2.reshape(1, H)
    b3r = b3.reshape(1, H)
    blr = b_lin.reshape(1, C)
    batch2 = batch.reshape(N // BN, 1, BN)

    yr1 = _pre(x, W1_rel, W1_root, b1r)
    p1 = _seg_sum(yr1.reshape(2 * N, H), src, dst)
    yr2 = _mid(p1, yr1, W2_rel, W2_root, b2r)
    p2 = _seg_sum(yr2.reshape(2 * N, H), src, dst)
    yr3 = _mid(p2, yr2, W3_rel, W3_root, b3r)
    p3 = _seg_sum(yr3.reshape(2 * N, H), src, dst)
    return _fin(p3, yr3, batch2, W_lin, blr)


# speedup vs baseline: 2.0080x; 2.0080x over previous
import jax
import jax.numpy as jnp
from jax import lax
from jax.experimental import pallas as pl
from jax.experimental.pallas import tpu as pltpu
from jax.experimental.pallas import tpu_sc as plsc

N = 10000
E = 320000
D = 128
H = 64
C = 10
G = 64

NC = 2
NS = 16
NW = NC * NS
EPW = E // NW
K = 100
EPAD = E
NCHUNK = EPAD // (NW * K)
NPAD = 10240
RPS = NPAD // NS

BN = 5000


NBUF = 8
GDEPTH = 4
ZROWS = 80


def _seg_sum_body(y_hbm, src_hbm, dst_hbm, out_hbm,
                  src_big, dst_big, rows, buf_v, acc_sh, gs, ss):
    c = lax.axis_index("c")
    s = lax.axis_index("s")
    wid = s * NC + c
    rbase = s * RPS

    zv = jnp.zeros((16,), jnp.float32)

    def zbody(r, carry):
        for u in range(H // 16):
            buf_v[r, pl.ds(16 * u, 16)] = zv
        return carry

    lax.fori_loop(0, ZROWS, zbody, 0)
    for zz in range(RPS // ZROWS):
        pltpu.sync_copy(buf_v, acc_sh.at[pl.ds(rbase + zz * ZROWS, ZROWS)])

    pltpu.sync_copy(src_hbm.at[pl.ds(wid * NCHUNK, NCHUNK)], src_big)
    pltpu.sync_copy(dst_hbm.at[pl.ds(wid * NCHUNK, NCHUNK)], dst_big)
    plsc.subcore_barrier()

    def gather(ck, b):
        pltpu.async_copy(y_hbm.at[src_big.at[ck]], rows[b], gs[b])

    def wait_gather(b):
        pltpu.make_async_copy(y_hbm.at[src_big.at[0]], rows[b], gs[b]).wait()

    def scatter(ck, b):
        pltpu.async_copy(rows[b], acc_sh.at[dst_big.at[ck]], ss[b], add=True)

    def wait_scatter(b):
        pltpu.make_async_copy(rows[b], acc_sh.at[dst_big.at[0]], ss[b]).wait()

    def full_step(cc, b):
        wait_gather(b)
        scatter(cc, b)
        bp = (b + GDEPTH) % NBUF
        wait_scatter(bp)
        gather(cc + GDEPTH, bp)

    def step(cc, b):
        wait_gather(b)
        scatter(cc, b)
        if cc + GDEPTH < NCHUNK:
            bp = (b + GDEPTH) % NBUF
            if cc + GDEPTH >= NBUF:
                wait_scatter(bp)
            gather(cc + GDEPTH, bp)

    for b in range(GDEPTH):
        gather(b, b)
    for cc in range(NBUF):
        step(cc, cc)

    n_loop = (NCHUNK - GDEPTH - NBUF) // NBUF

    def body(j, carry):
        c0 = j * NBUF
        for u in range(NBUF):
            full_step(c0 + u, u)
        return carry

    lax.fori_loop(1, 1 + n_loop, body, 0)

    for cc in range(NBUF + n_loop * NBUF, NCHUNK):
        step(cc, cc % NBUF)
    for b in range(NBUF):
        wait_scatter(b)
    plsc.subcore_barrier()

    ov = out_hbm.at[:, pl.ds(H * c, H)]
    for zz in range(RPS // ZROWS):
        pltpu.sync_copy(acc_sh.at[pl.ds(rbase + zz * ZROWS, ZROWS)], buf_v)
        pltpu.sync_copy(buf_v, ov.at[pl.ds(rbase + zz * ZROWS, ZROWS)])


def _seg_sum(y, src2, dst2):
    f = pl.kernel(
        _seg_sum_body,
        out_type=jax.ShapeDtypeStruct((NPAD, 2 * H), jnp.float32),
        mesh=plsc.VectorSubcoreMesh(core_axis_name="c", subcore_axis_name="s",
                                    num_cores=NC, num_subcores=NS),
        scratch_types=[
            pltpu.VMEM((NCHUNK, K), jnp.int32),
            pltpu.VMEM((NCHUNK, K), jnp.int32),
            [pltpu.VMEM((K, H), jnp.float32) for _ in range(NBUF)],
            pltpu.VMEM((ZROWS, H), jnp.float32),
            pltpu.VMEM_SHARED((NPAD, H), jnp.float32),
            [pltpu.SemaphoreType.DMA for _ in range(NBUF)],
            [pltpu.SemaphoreType.DMA for _ in range(NBUF)],
        ],
        compiler_params=pltpu.CompilerParams(use_tc_tiling_on_sc=False),
    )
    return f(y, src2, dst2)


def _dot_t(a, w):
    return lax.dot_general(a, w, (((1,), (1,)), ((), ())),
                           preferred_element_type=jnp.float32)


def _pre_body(x_ref, wr_ref, wo_ref, b_ref, yr_ref):
    x = x_ref[...]
    y = _dot_t(x, wr_ref[...])
    r = _dot_t(x, wo_ref[...]) + b_ref[...]
    yr_ref[...] = jnp.concatenate([y, r], axis=1)


_pre = pl.pallas_call(
    _pre_body,
    grid=(N // BN,),
    in_specs=[
        pl.BlockSpec((BN, D), lambda i: (i, 0)),
        pl.BlockSpec((H, D), lambda i: (0, 0)),
        pl.BlockSpec((H, D), lambda i: (0, 0)),
        pl.BlockSpec((1, H), lambda i: (0, 0)),
    ],
    out_specs=pl.BlockSpec((BN, 2 * H), lambda i: (i, 0)),
    out_shape=jax.ShapeDtypeStruct((N, 2 * H), jnp.float32),
)


def _mid_body(p_ref, yr_ref, wr_ref, wo_ref, b_ref, yrn_ref):
    p = p_ref[...]
    h = jnp.maximum(p[:, :H] + p[:, H:] + yr_ref[...][:, H:], 0.0)
    y = _dot_t(h, wr_ref[...])
    rn = _dot_t(h, wo_ref[...]) + b_ref[...]
    yrn_ref[...] = jnp.concatenate([y, rn], axis=1)


_mid = pl.pallas_call(
    _mid_body,
    grid=(N // BN,),
    in_specs=[
        pl.BlockSpec((BN, 2 * H), lambda i: (i, 0)),
        pl.BlockSpec((BN, 2 * H), lambda i: (i, 0)),
        pl.BlockSpec((H, H), lambda i: (0, 0)),
        pl.BlockSpec((H, H), lambda i: (0, 0)),
        pl.BlockSpec((1, H), lambda i: (0, 0)),
    ],
    out_specs=pl.BlockSpec((BN, 2 * H), lambda i: (i, 0)),
    out_shape=jax.ShapeDtypeStruct((N, 2 * H), jnp.float32),
)


def _fin_body(p_ref, yr_ref, batch_ref, wl_ref, bl_ref, out_ref,
              acc_ref, cnt_ref):
    i = pl.program_id(0)

    @pl.when(i == 0)
    def _():
        acc_ref[...] = jnp.zeros_like(acc_ref)
        cnt_ref[...] = jnp.zeros_like(cnt_ref)

    p = p_ref[...]
    h = p[:, :H] + p[:, H:] + yr_ref[...][:, H:]
    b = batch_ref[...][0]
    gi = lax.broadcasted_iota(jnp.int32, (G, BN), 0)
    onehot = (gi == b).astype(jnp.float32)
    acc_ref[...] += lax.dot_general(
        onehot, h, (((1,), (0,)), ((), ())),
        preferred_element_type=jnp.float32)
    cnt_ref[...] += jnp.sum(onehot, axis=1, keepdims=True)

    @pl.when(i == pl.num_programs(0) - 1)
    def _():
        pool = acc_ref[...] / jnp.maximum(cnt_ref[...], 1.0)
        out_ref[...] = _dot_t(pool, wl_ref[...]) + bl_ref[...]


_fin = pl.pallas_call(
    _fin_body,
    grid=(N // BN,),
    in_specs=[
        pl.BlockSpec((BN, 2 * H), lambda i: (i, 0)),
        pl.BlockSpec((BN, 2 * H), lambda i: (i, 0)),
        pl.BlockSpec((1, 1, BN), lambda i: (i, 0, 0)),
        pl.BlockSpec((C, H), lambda i: (0, 0)),
        pl.BlockSpec((1, C), lambda i: (0, 0)),
    ],
    out_specs=pl.BlockSpec((G, C), lambda i: (0, 0)),
    out_shape=jax.ShapeDtypeStruct((G, C), jnp.float32),
    scratch_shapes=[pltpu.VMEM((G, H), jnp.float32),
                    pltpu.VMEM((G, 1), jnp.float32)],
)


def kernel(x, edge_index, batch, W1_rel, b1, W1_root, W2_rel, b2, W2_root,
           W3_rel, b3, W3_root, W_lin, b_lin):
    src = (edge_index[0] * 2).reshape(EPAD // K, K)
    dst = edge_index[1].reshape(EPAD // K, K)
    b1r = b1.reshape(1, H)
    b2r = b2.reshape(1, H)
    b3r = b3.reshape(1, H)
    blr = b_lin.reshape(1, C)
    batch2 = batch.reshape(N // BN, 1, BN)

    yr1 = _pre(x, W1_rel, W1_root, b1r)
    p1 = _seg_sum(yr1.reshape(2 * N, H), src, dst)
    yr2 = _mid(p1, yr1, W2_rel, W2_root, b2r)
    p2 = _seg_sum(yr2.reshape(2 * N, H), src, dst)
    yr3 = _mid(p2, yr2, W3_rel, W3_root, b3r)
    p3 = _seg_sum(yr3.reshape(2 * N, H), src, dst)
    return _fin(p3, yr3, batch2, W_lin, blr)
